# Initial kernel scaffold; baseline (speedup 1.0000x reference)
import jax
import jax.numpy as jnp
from jax.experimental import pallas as pl


def kernel(floor_position, floor_normal, floor_z_value, wall_position, wall_normal, trans_object_obb, trans_object_abb, trans_object_obb_center, translate, euler_angle, scale, trans_object_obb_center_dist, trans_object_abb_eiou, obj_w1, obj_b1, obj_w2, obj_b2, rel_w1, rel_b1, rel_w2, rel_b2, floor_w1, floor_b1, floor_w2, floor_b2, wall_w1, wall_b1, wall_w2, wall_b2, wc_rel_sub, wc_rel_obj, wc_obj_sub, wc_obj_obj, wu_obj, wu_rel, tr_w1, tr_b1, tr_w2, tr_b2, eu_w1, eu_b1, eu_w2, eu_b2, sc_w1, sc_b1, sc_w2, sc_b2):
    raise NotImplementedError("write your pallas kernel here")


# speedup vs baseline: 10.6679x; 10.6679x over previous
import functools

import jax
import jax.numpy as jnp
from jax import lax
from jax.experimental import pallas as pl
from jax.experimental.pallas import tpu as pltpu

O_NUM, W_NUM, F_NUM = 200, 40, 16
T_NUM = O_NUM + W_NUM + F_NUM
FDIM = 512
STEPS = 4
DEG = float(T_NUM - 1)

RB = 8
GRID = T_NUM // RB
BLK = RB * T_NUM
REAL_BLOCKS = O_NUM // RB
PAD_VAL = 0.001

F32 = jnp.float32


def _dot(a, b):
    return jnp.dot(a, b, preferred_element_type=F32)


def _node_enc_kernel(ox, ow1, ob1, ow2, ob2,
                     wx, ww1, wb1, ww2, wb2,
                     fx, fw1, fb1, fw2, fb2, out_ref):
    h = jnp.maximum(_dot(ox[...], ow1[...]) + ob1[...], 0.0)
    out_ref[0:O_NUM, :] = _dot(h, ow2[...]) + ob2[...]
    h = jnp.maximum(_dot(wx[...], ww1[...]) + wb1[...], 0.0)
    out_ref[O_NUM:O_NUM + W_NUM, :] = _dot(h, ww2[...]) + wb2[...]
    h = jnp.maximum(_dot(fx[...], fw1[...]) + fb1[...], 0.0)
    out_ref[O_NUM + W_NUM:T_NUM, :] = _dot(h, fw2[...]) + fb2[...]


def _masked_reduce(e2, pid, wcs_ref, wco_ref, msub_ref, mobj_ref, acc_ref):
    p_s = jnp.maximum(_dot(e2, wcs_ref[...]), 0.0)
    p_o = jnp.maximum(_dot(e2, wco_ref[...]), 0.0)
    q = lax.broadcasted_iota(jnp.int32, (BLK, FDIM), 0)
    dmask = (q & (T_NUM - 1)) == (q >> 8) + pid * RB
    p_s = jnp.where(dmask, 0.0, p_s)
    p_o = jnp.where(dmask, 0.0, p_o)
    rows = [p_s[r * T_NUM:(r + 1) * T_NUM, :].sum(axis=0, keepdims=True)
            for r in range(RB)]
    msub_ref[...] = jnp.concatenate(rows, axis=0)
    colsum = p_o[0:T_NUM, :]
    for r in range(1, RB):
        colsum = colsum + p_o[r * T_NUM:(r + 1) * T_NUM, :]

    @pl.when(pid == 0)
    def _():
        acc_ref[...] = colsum

    @pl.when(pid > 0)
    def _():
        acc_ref[...] = acc_ref[...] + colsum

    @pl.when(pid == GRID - 1)
    def _():
        mobj_ref[...] = acc_ref[...]


def _build_reduce_kernel(rin_ref, w1, b1, w2, b2, wcs, wco,
                         rel_ref, msub_ref, mobj_ref, acc_ref):
    pid = pl.program_id(0)

    @pl.when(pid < REAL_BLOCKS)
    def _():
        x = rin_ref[0]
        h = jnp.maximum(_dot(x, w1[...]) + b1[...], 0.0)
        y = _dot(h, w2[...]) + b2[...]
        pad = jnp.full((T_NUM - O_NUM, FDIM), PAD_VAL, F32)
        parts = []
        for r in range(RB):
            parts.append(y[r * O_NUM:(r + 1) * O_NUM, :])
            parts.append(pad)
        rel_ref[...] = jnp.concatenate(parts, axis=0)

    @pl.when(pid >= REAL_BLOCKS)
    def _():
        rel_ref[...] = jnp.full((BLK, FDIM), PAD_VAL, F32)

    _masked_reduce(rel_ref[...], pid, wcs, wco, msub_ref, mobj_ref, acc_ref)


def _node_update_kernel(nf, msub, mobj, wuo, wcs_o, wco_o, wur,
                        nf_out, a_out, b_out):
    m = (msub[...] + mobj[...]) * (0.5 / DEG)
    n2 = jnp.maximum(nf[...] + _dot(m, wuo[...]), 0.0)
    nf_out[...] = n2
    a_out[...] = _dot(jnp.maximum(_dot(n2, wcs_o[...]), 0.0), wur[...]) * 0.5
    b_out[...] = _dot(jnp.maximum(_dot(n2, wco_o[...]), 0.0), wur[...]) * 0.5


def _fused_update_reduce_kernel(relp, a_ref, b_ref, wcs, wco,
                                reln, msub_ref, mobj_ref, acc_ref,
                                *, write_rel):
    pid = pl.program_id(0)
    e2 = relp[...]
    b = b_ref[...]
    parts = []
    for r in range(RB):
        seg = e2[r * T_NUM:(r + 1) * T_NUM, :] + a_ref[r:r + 1, :] + b
        parts.append(jnp.maximum(seg, 0.0))
    en2 = jnp.concatenate(parts, axis=0)
    if write_rel:
        reln[...] = en2
    _masked_reduce(en2, pid, wcs, wco, msub_ref, mobj_ref, acc_ref)


def _out_enc_kernel(nf, tw1, tb1, tw2, tb2, ew1, eb1, ew2, eb2,
                    sw1, sb1, sw2, sb2, t_out, e_out, s_out):
    x = nf[0:O_NUM, :]
    for w1, b1, w2, b2, o in ((tw1, tb1, tw2, tb2, t_out),
                              (ew1, eb1, ew2, eb2, e_out),
                              (sw1, sb1, sw2, sb2, s_out)):
        h = _dot(x, w1[...]) + b1[...]
        h = jnp.where(h > 0, h, 0.2 * h)
        o[...] = _dot(h, w2[...]) + b2[...]


def _pad2(x, r, c):
    return jnp.pad(x, ((0, r - x.shape[0]), (0, c - x.shape[1])))


def kernel(floor_position, floor_normal, floor_z_value, wall_position,
           wall_normal, trans_object_obb, trans_object_abb,
           trans_object_obb_center, translate, euler_angle, scale,
           trans_object_obb_center_dist, trans_object_abb_eiou,
           obj_w1, obj_b1, obj_w2, obj_b2,
           rel_w1, rel_b1, rel_w2, rel_b2,
           floor_w1, floor_b1, floor_w2, floor_b2,
           wall_w1, wall_b1, wall_w2, wall_b2,
           wc_rel_sub, wc_rel_obj, wc_obj_sub, wc_obj_obj, wu_obj, wu_rel,
           tr_w1, tr_b1, tr_w2, tr_b2,
           eu_w1, eu_b1, eu_w2, eu_b2,
           sc_w1, sc_b1, sc_w2, sc_b2):
    obj_in = jnp.concatenate(
        [trans_object_obb, trans_object_abb, trans_object_obb_center,
         translate, euler_angle, scale], axis=-1)[0]
    wall_in = jnp.concatenate([wall_position, wall_normal], axis=-1)[0]
    floor_in = jnp.concatenate(
        [floor_position, floor_normal, floor_z_value], axis=-1)[0]
    rel_in = jnp.concatenate(
        [trans_object_obb_center_dist, trans_object_abb_eiou], axis=-1)[0]

    obj_x = _pad2(obj_in, O_NUM, 128)
    wall_x = _pad2(wall_in, W_NUM, 128)
    floor_x = _pad2(floor_in, F_NUM, 128)
    rin3 = _pad2(rel_in, O_NUM * O_NUM, 8).reshape(REAL_BLOCKS, RB * O_NUM, 8)

    ow1 = _pad2(obj_w1, 128, FDIM)
    ww1 = _pad2(wall_w1, 128, FDIM)
    fw1 = _pad2(floor_w1, 128, FDIM)
    rw1 = _pad2(rel_w1, 8, FDIM)

    def row(b):
        return b.reshape(1, -1)

    nsd = jax.ShapeDtypeStruct((T_NUM, FDIM), F32)

    node_f = pl.pallas_call(_node_enc_kernel, out_shape=nsd)(
        obj_x, ow1, row(obj_b1), obj_w2, row(obj_b2),
        wall_x, ww1, row(wall_b1), wall_w2, row(wall_b2),
        floor_x, fw1, row(floor_b1), floor_w2, row(floor_b2))

    full = lambda shape: pl.BlockSpec(shape, lambda i: (0,) * len(shape))
    blk_spec = pl.BlockSpec((BLK, FDIM), lambda i: (i, 0))
    msub_spec = pl.BlockSpec((RB, FDIM), lambda i: (i, 0))
    scratch = [pltpu.VMEM((T_NUM, FDIM), F32)]

    rel0, msub, mobj = pl.pallas_call(
        _build_reduce_kernel,
        grid=(GRID,),
        in_specs=[
            pl.BlockSpec((1, RB * O_NUM, 8),
                         lambda i: (jnp.minimum(i, REAL_BLOCKS - 1), 0, 0)),
            full((8, FDIM)), full((1, FDIM)), full((FDIM, FDIM)),
            full((1, FDIM)), full((FDIM, FDIM)), full((FDIM, FDIM)),
        ],
        out_specs=[blk_spec, msub_spec, full((T_NUM, FDIM))],
        out_shape=[jax.ShapeDtypeStruct((T_NUM * T_NUM, FDIM), F32), nsd, nsd],
        scratch_shapes=scratch,
    )(rin3, rw1, row(rel_b1), rel_w2, row(rel_b2), wc_rel_sub, wc_rel_obj)

    node_upd = pl.pallas_call(_node_update_kernel,
                              out_shape=[nsd, nsd, nsd])

    def fused(write_rel):
        body = functools.partial(_fused_update_reduce_kernel,
                                 write_rel=write_rel)
        if not write_rel:
            def body(relp, a_ref, b_ref, wcs, wco, msub_ref, mobj_ref,
                     acc_ref):
                _fused_update_reduce_kernel(
                    relp, a_ref, b_ref, wcs, wco, None, msub_ref, mobj_ref,
                    acc_ref, write_rel=False)
        out_specs = [msub_spec, full((T_NUM, FDIM))]
        out_shape = [nsd, nsd]
        if write_rel:
            out_specs = [blk_spec] + out_specs
            out_shape = [jax.ShapeDtypeStruct((T_NUM * T_NUM, FDIM), F32)] \
                + out_shape
        return pl.pallas_call(
            body,
            grid=(GRID,),
            in_specs=[blk_spec, msub_spec, full((T_NUM, FDIM)),
                      full((FDIM, FDIM)), full((FDIM, FDIM))],
            out_specs=out_specs,
            out_shape=out_shape,
            scratch_shapes=scratch,
        )

    fused_w = fused(True)
    fused_last = fused(False)

    rel = rel0
    for step in range(STEPS):
        node_f, a_vec, b_vec = node_upd(
            node_f, msub, mobj, wu_obj, wc_obj_sub, wc_obj_obj, wu_rel)
        if step == STEPS - 1:
            break
        if step < STEPS - 2:
            rel, msub, mobj = fused_w(
                rel, a_vec, b_vec, wc_rel_sub, wc_rel_obj)
        else:
            msub, mobj = fused_last(
                rel, a_vec, b_vec, wc_rel_sub, wc_rel_obj)

    osd = jax.ShapeDtypeStruct((O_NUM, 128), F32)
    tr, eu, sc = pl.pallas_call(_out_enc_kernel, out_shape=[osd, osd, osd])(
        node_f,
        tr_w1, row(tr_b1), _pad2(tr_w2, FDIM // 2, 128), row(_pad2(tr_b2.reshape(1, -1), 1, 128)[0]),
        eu_w1, row(eu_b1), _pad2(eu_w2, FDIM // 2, 128), row(_pad2(eu_b2.reshape(1, -1), 1, 128)[0]),
        sc_w1, row(sc_b1), _pad2(sc_w2, FDIM // 2, 128), row(_pad2(sc_b2.reshape(1, -1), 1, 128)[0]))
    return jnp.concatenate([tr[:, :3], eu[:, :3], sc[:, :3]], axis=-1)


# speedup vs baseline: 11.2652x; 1.0560x over previous
import functools

import jax
import jax.numpy as jnp
from jax import lax
from jax.experimental import pallas as pl
from jax.experimental.pallas import tpu as pltpu

O_NUM, W_NUM, F_NUM = 200, 40, 16
T_NUM = O_NUM + W_NUM + F_NUM
FDIM = 512
STEPS = 4
DEG = float(T_NUM - 1)

RB = 8
GRID = T_NUM // RB
BLK = RB * T_NUM
REAL_BLOCKS = O_NUM // RB
PAD_VAL = 0.001

F32 = jnp.float32
BF16 = jnp.bfloat16


def _dot(a, b):
    return jnp.dot(a, b, preferred_element_type=F32)


def _node_enc_kernel(ox, ow1, ob1, ow2, ob2,
                     wx, ww1, wb1, ww2, wb2,
                     fx, fw1, fb1, fw2, fb2, out_ref):
    h = jnp.maximum(_dot(ox[...], ow1[...]) + ob1[...], 0.0)
    out_ref[0:O_NUM, :] = _dot(h, ow2[...]) + ob2[...]
    h = jnp.maximum(_dot(wx[...], ww1[...]) + wb1[...], 0.0)
    out_ref[O_NUM:O_NUM + W_NUM, :] = _dot(h, ww2[...]) + wb2[...]
    h = jnp.maximum(_dot(fx[...], fw1[...]) + fb1[...], 0.0)
    out_ref[O_NUM + W_NUM:T_NUM, :] = _dot(h, fw2[...]) + fb2[...]


def _masked_reduce(e2, pid, wcs_ref, wco_ref, msub_ref, mobj_ref, acc_ref):
    p_s = jnp.maximum(_dot(e2, wcs_ref[...]), 0.0)
    p_o = jnp.maximum(_dot(e2, wco_ref[...]), 0.0)
    q = lax.broadcasted_iota(jnp.int32, (BLK, FDIM), 0)
    dmask = (q & (T_NUM - 1)) == (q >> 8) + pid * RB
    p_s = jnp.where(dmask, 0.0, p_s)
    p_o = jnp.where(dmask, 0.0, p_o)
    rows = [p_s[r * T_NUM:(r + 1) * T_NUM, :].sum(axis=0, keepdims=True)
            for r in range(RB)]
    msub_ref[...] = jnp.concatenate(rows, axis=0)
    colsum = p_o[0:T_NUM, :]
    for r in range(1, RB):
        colsum = colsum + p_o[r * T_NUM:(r + 1) * T_NUM, :]

    @pl.when(pid == 0)
    def _():
        acc_ref[...] = colsum

    @pl.when(pid > 0)
    def _():
        acc_ref[...] = acc_ref[...] + colsum

    @pl.when(pid == GRID - 1)
    def _():
        mobj_ref[...] = acc_ref[...]


def _build_reduce_kernel(rin_ref, w1, b1, w2, b2, wcs, wco,
                         rel_ref, msub_ref, mobj_ref, acc_ref):
    pid = pl.program_id(0)

    @pl.when(pid < REAL_BLOCKS)
    def _():
        x = rin_ref[0]
        h = jnp.maximum(_dot(x, w1[...]) + b1[...], 0.0)
        y = _dot(h.astype(BF16), w2[...]) + b2[...]
        pad = jnp.full((T_NUM - O_NUM, FDIM), PAD_VAL, F32)
        parts = []
        for r in range(RB):
            parts.append(y[r * O_NUM:(r + 1) * O_NUM, :])
            parts.append(pad)
        rel_ref[...] = jnp.concatenate(parts, axis=0).astype(BF16)

    @pl.when(pid >= REAL_BLOCKS)
    def _():
        rel_ref[...] = jnp.full((BLK, FDIM), PAD_VAL, BF16)

    _masked_reduce(rel_ref[...], pid, wcs, wco, msub_ref, mobj_ref, acc_ref)


def _node_update_kernel(nf, msub, mobj, wuo, wcs_o, wco_o, wur,
                        nf_out, a_out, b_out):
    m = (msub[...] + mobj[...]) * (0.5 / DEG)
    n2 = jnp.maximum(nf[...] + _dot(m, wuo[...]), 0.0)
    nf_out[...] = n2
    a_out[...] = _dot(jnp.maximum(_dot(n2, wcs_o[...]), 0.0), wur[...]) * 0.5
    b_out[...] = _dot(jnp.maximum(_dot(n2, wco_o[...]), 0.0), wur[...]) * 0.5


def _fused_update_reduce_kernel(relp, a_ref, b_ref, wcs, wco,
                                reln, msub_ref, mobj_ref, acc_ref,
                                *, write_rel):
    pid = pl.program_id(0)
    e2 = relp[...]
    b = b_ref[...]
    parts = []
    for r in range(RB):
        seg = e2[r * T_NUM:(r + 1) * T_NUM, :].astype(F32) \
            + a_ref[r:r + 1, :] + b
        parts.append(jnp.maximum(seg, 0.0))
    en2 = jnp.concatenate(parts, axis=0).astype(BF16)
    if write_rel:
        reln[...] = en2
    _masked_reduce(en2, pid, wcs, wco, msub_ref, mobj_ref, acc_ref)


def _out_enc_kernel(nf, tw1, tb1, tw2, tb2, ew1, eb1, ew2, eb2,
                    sw1, sb1, sw2, sb2, t_out, e_out, s_out):
    x = nf[0:O_NUM, :]
    for w1, b1, w2, b2, o in ((tw1, tb1, tw2, tb2, t_out),
                              (ew1, eb1, ew2, eb2, e_out),
                              (sw1, sb1, sw2, sb2, s_out)):
        h = _dot(x, w1[...]) + b1[...]
        h = jnp.where(h > 0, h, 0.2 * h)
        o[...] = _dot(h, w2[...]) + b2[...]


def _pad2(x, r, c):
    return jnp.pad(x, ((0, r - x.shape[0]), (0, c - x.shape[1])))


def kernel(floor_position, floor_normal, floor_z_value, wall_position,
           wall_normal, trans_object_obb, trans_object_abb,
           trans_object_obb_center, translate, euler_angle, scale,
           trans_object_obb_center_dist, trans_object_abb_eiou,
           obj_w1, obj_b1, obj_w2, obj_b2,
           rel_w1, rel_b1, rel_w2, rel_b2,
           floor_w1, floor_b1, floor_w2, floor_b2,
           wall_w1, wall_b1, wall_w2, wall_b2,
           wc_rel_sub, wc_rel_obj, wc_obj_sub, wc_obj_obj, wu_obj, wu_rel,
           tr_w1, tr_b1, tr_w2, tr_b2,
           eu_w1, eu_b1, eu_w2, eu_b2,
           sc_w1, sc_b1, sc_w2, sc_b2):
    obj_in = jnp.concatenate(
        [trans_object_obb, trans_object_abb, trans_object_obb_center,
         translate, euler_angle, scale], axis=-1)[0]
    wall_in = jnp.concatenate([wall_position, wall_normal], axis=-1)[0]
    floor_in = jnp.concatenate(
        [floor_position, floor_normal, floor_z_value], axis=-1)[0]
    rel_in = jnp.concatenate(
        [trans_object_obb_center_dist, trans_object_abb_eiou], axis=-1)[0]

    obj_x = _pad2(obj_in, O_NUM, 128)
    wall_x = _pad2(wall_in, W_NUM, 128)
    floor_x = _pad2(floor_in, F_NUM, 128)
    rin3 = _pad2(rel_in, O_NUM * O_NUM, 8).reshape(REAL_BLOCKS, RB * O_NUM, 8)

    ow1 = _pad2(obj_w1, 128, FDIM)
    ww1 = _pad2(wall_w1, 128, FDIM)
    fw1 = _pad2(floor_w1, 128, FDIM)
    rw1 = _pad2(rel_w1, 8, FDIM)

    def row(b):
        return b.reshape(1, -1)

    nsd = jax.ShapeDtypeStruct((T_NUM, FDIM), F32)
    rel_sd = jax.ShapeDtypeStruct((T_NUM * T_NUM, FDIM), BF16)
    wcs_bf = wc_rel_sub.astype(BF16)
    wco_bf = wc_rel_obj.astype(BF16)
    rw2_bf = rel_w2.astype(BF16)

    node_f = pl.pallas_call(_node_enc_kernel, out_shape=nsd)(
        obj_x, ow1, row(obj_b1), obj_w2, row(obj_b2),
        wall_x, ww1, row(wall_b1), wall_w2, row(wall_b2),
        floor_x, fw1, row(floor_b1), floor_w2, row(floor_b2))

    full = lambda shape: pl.BlockSpec(shape, lambda i: (0,) * len(shape))
    blk_spec = pl.BlockSpec((BLK, FDIM), lambda i: (i, 0))
    msub_spec = pl.BlockSpec((RB, FDIM), lambda i: (i, 0))
    scratch = [pltpu.VMEM((T_NUM, FDIM), F32)]

    rel0, msub, mobj = pl.pallas_call(
        _build_reduce_kernel,
        grid=(GRID,),
        in_specs=[
            pl.BlockSpec((1, RB * O_NUM, 8),
                         lambda i: (jnp.minimum(i, REAL_BLOCKS - 1), 0, 0)),
            full((8, FDIM)), full((1, FDIM)), full((FDIM, FDIM)),
            full((1, FDIM)), full((FDIM, FDIM)), full((FDIM, FDIM)),
        ],
        out_specs=[blk_spec, msub_spec, full((T_NUM, FDIM))],
        out_shape=[rel_sd, nsd, nsd],
        scratch_shapes=scratch,
    )(rin3, rw1, row(rel_b1), rw2_bf, row(rel_b2), wcs_bf, wco_bf)

    node_upd = pl.pallas_call(_node_update_kernel,
                              out_shape=[nsd, nsd, nsd])

    def fused(write_rel):
        body = functools.partial(_fused_update_reduce_kernel,
                                 write_rel=write_rel)
        if not write_rel:
            def body(relp, a_ref, b_ref, wcs, wco, msub_ref, mobj_ref,
                     acc_ref):
                _fused_update_reduce_kernel(
                    relp, a_ref, b_ref, wcs, wco, None, msub_ref, mobj_ref,
                    acc_ref, write_rel=False)
        out_specs = [msub_spec, full((T_NUM, FDIM))]
        out_shape = [nsd, nsd]
        if write_rel:
            out_specs = [blk_spec] + out_specs
            out_shape = [rel_sd] + out_shape
        return pl.pallas_call(
            body,
            grid=(GRID,),
            in_specs=[blk_spec, msub_spec, full((T_NUM, FDIM)),
                      full((FDIM, FDIM)), full((FDIM, FDIM))],
            out_specs=out_specs,
            out_shape=out_shape,
            scratch_shapes=scratch,
        )

    fused_w = fused(True)
    fused_last = fused(False)

    rel = rel0
    for step in range(STEPS):
        node_f, a_vec, b_vec = node_upd(
            node_f, msub, mobj, wu_obj, wc_obj_sub, wc_obj_obj, wu_rel)
        if step == STEPS - 1:
            break
        if step < STEPS - 2:
            rel, msub, mobj = fused_w(rel, a_vec, b_vec, wcs_bf, wco_bf)
        else:
            msub, mobj = fused_last(rel, a_vec, b_vec, wcs_bf, wco_bf)

    osd = jax.ShapeDtypeStruct((O_NUM, 128), F32)
    tr, eu, sc = pl.pallas_call(_out_enc_kernel, out_shape=[osd, osd, osd])(
        node_f,
        tr_w1, row(tr_b1), _pad2(tr_w2, FDIM // 2, 128), row(_pad2(tr_b2.reshape(1, -1), 1, 128)[0]),
        eu_w1, row(eu_b1), _pad2(eu_w2, FDIM // 2, 128), row(_pad2(eu_b2.reshape(1, -1), 1, 128)[0]),
        sc_w1, row(sc_b1), _pad2(sc_w2, FDIM // 2, 128), row(_pad2(sc_b2.reshape(1, -1), 1, 128)[0]))
    return jnp.concatenate([tr[:, :3], eu[:, :3], sc[:, :3]], axis=-1)


# speedup vs baseline: 11.7393x; 1.0421x over previous
import functools

import jax
import jax.numpy as jnp
from jax import lax
from jax.experimental import pallas as pl
from jax.experimental.pallas import tpu as pltpu

O_NUM, W_NUM, F_NUM = 200, 40, 16
T_NUM = O_NUM + W_NUM + F_NUM
FDIM = 512
STEPS = 4
DEG = float(T_NUM - 1)

RB = 8
GRID = T_NUM // RB
BLK = RB * T_NUM
RBF = 16
GRIDF = T_NUM // RBF
BLKF = RBF * T_NUM
REAL_BLOCKS = O_NUM // RB
PAD_VAL = 0.001

F32 = jnp.float32
BF16 = jnp.bfloat16


def _dot(a, b):
    return jnp.dot(a, b, preferred_element_type=F32)


def _node_enc_kernel(ox, ow1, ob1, ow2, ob2,
                     wx, ww1, wb1, ww2, wb2,
                     fx, fw1, fb1, fw2, fb2, out_ref):
    h = jnp.maximum(_dot(ox[...], ow1[...]) + ob1[...], 0.0)
    out_ref[0:O_NUM, :] = _dot(h, ow2[...]) + ob2[...]
    h = jnp.maximum(_dot(wx[...], ww1[...]) + wb1[...], 0.0)
    out_ref[O_NUM:O_NUM + W_NUM, :] = _dot(h, ww2[...]) + wb2[...]
    h = jnp.maximum(_dot(fx[...], fw1[...]) + fb1[...], 0.0)
    out_ref[O_NUM + W_NUM:T_NUM, :] = _dot(h, fw2[...]) + fb2[...]


def _masked_reduce(e2, pid, wcs_ref, wco_ref, msub_ref, mobj_ref, acc_ref,
                   rb, grid):
    p_s = jnp.maximum(_dot(e2, wcs_ref[...]), 0.0)
    p_o = jnp.maximum(_dot(e2, wco_ref[...]), 0.0)
    q = lax.broadcasted_iota(jnp.int32, (rb * T_NUM, FDIM), 0)
    dmask = (q & (T_NUM - 1)) == (q >> 8) + pid * rb
    p_s = jnp.where(dmask, 0.0, p_s)
    p_o = jnp.where(dmask, 0.0, p_o)
    rows = [p_s[r * T_NUM:(r + 1) * T_NUM, :].sum(axis=0, keepdims=True)
            for r in range(rb)]
    msub_ref[...] = jnp.concatenate(rows, axis=0)
    colsum = p_o[0:T_NUM, :]
    for r in range(1, rb):
        colsum = colsum + p_o[r * T_NUM:(r + 1) * T_NUM, :]

    @pl.when(pid == 0)
    def _():
        acc_ref[...] = colsum

    @pl.when(pid > 0)
    def _():
        acc_ref[...] = acc_ref[...] + colsum

    @pl.when(pid == grid - 1)
    def _():
        mobj_ref[...] = acc_ref[...]


def _build_reduce_kernel(rin_ref, w1, b1, w2, b2, wcs, wco,
                         rel_ref, msub_ref, mobj_ref, acc_ref):
    pid = pl.program_id(0)

    @pl.when(pid < REAL_BLOCKS)
    def _():
        x = rin_ref[0]
        h = jnp.maximum(_dot(x, w1[...]) + b1[...], 0.0)
        y = _dot(h.astype(BF16), w2[...]) + b2[...]
        pad = jnp.full((T_NUM - O_NUM, FDIM), PAD_VAL, F32)
        parts = []
        for r in range(RB):
            parts.append(y[r * O_NUM:(r + 1) * O_NUM, :])
            parts.append(pad)
        rel_ref[...] = jnp.concatenate(parts, axis=0).astype(BF16)

    @pl.when(pid >= REAL_BLOCKS)
    def _():
        rel_ref[...] = jnp.full((BLK, FDIM), PAD_VAL, BF16)

    _masked_reduce(rel_ref[...], pid, wcs, wco, msub_ref, mobj_ref, acc_ref,
                   RB, GRID)


def _node_update_kernel(nf, msub, mobj, wuo, wcs_o, wco_o, wur,
                        nf_out, a_out, b_out):
    m = (msub[...] + mobj[...]) * (0.5 / DEG)
    n2 = jnp.maximum(nf[...] + _dot(m, wuo[...]), 0.0)
    nf_out[...] = n2
    a_out[...] = _dot(jnp.maximum(_dot(n2, wcs_o[...]), 0.0), wur[...]) * 0.5
    b_out[...] = _dot(jnp.maximum(_dot(n2, wco_o[...]), 0.0), wur[...]) * 0.5


def _fused_update_reduce_kernel(relp, a_ref, b_ref, wcs, wco,
                                reln, msub_ref, mobj_ref, acc_ref,
                                *, write_rel):
    pid = pl.program_id(0)
    e2 = relp[...]
    b = b_ref[...].astype(BF16)
    a = a_ref[...].astype(BF16)
    parts = []
    for r in range(RBF):
        seg = e2[r * T_NUM:(r + 1) * T_NUM, :] + a[r:r + 1, :] + b
        parts.append(jnp.maximum(seg, jnp.bfloat16(0.0)))
    en2 = jnp.concatenate(parts, axis=0)
    if write_rel:
        reln[...] = en2
    _masked_reduce(en2, pid, wcs, wco, msub_ref, mobj_ref, acc_ref,
                   RBF, GRIDF)


def _out_enc_kernel(nf, tw1, tb1, tw2, tb2, ew1, eb1, ew2, eb2,
                    sw1, sb1, sw2, sb2, t_out, e_out, s_out):
    x = nf[0:O_NUM, :]
    for w1, b1, w2, b2, o in ((tw1, tb1, tw2, tb2, t_out),
                              (ew1, eb1, ew2, eb2, e_out),
                              (sw1, sb1, sw2, sb2, s_out)):
        h = _dot(x, w1[...]) + b1[...]
        h = jnp.where(h > 0, h, 0.2 * h)
        o[...] = _dot(h, w2[...]) + b2[...]


def _pad2(x, r, c):
    return jnp.pad(x, ((0, r - x.shape[0]), (0, c - x.shape[1])))


def kernel(floor_position, floor_normal, floor_z_value, wall_position,
           wall_normal, trans_object_obb, trans_object_abb,
           trans_object_obb_center, translate, euler_angle, scale,
           trans_object_obb_center_dist, trans_object_abb_eiou,
           obj_w1, obj_b1, obj_w2, obj_b2,
           rel_w1, rel_b1, rel_w2, rel_b2,
           floor_w1, floor_b1, floor_w2, floor_b2,
           wall_w1, wall_b1, wall_w2, wall_b2,
           wc_rel_sub, wc_rel_obj, wc_obj_sub, wc_obj_obj, wu_obj, wu_rel,
           tr_w1, tr_b1, tr_w2, tr_b2,
           eu_w1, eu_b1, eu_w2, eu_b2,
           sc_w1, sc_b1, sc_w2, sc_b2):
    obj_in = jnp.concatenate(
        [trans_object_obb, trans_object_abb, trans_object_obb_center,
         translate, euler_angle, scale], axis=-1)[0]
    wall_in = jnp.concatenate([wall_position, wall_normal], axis=-1)[0]
    floor_in = jnp.concatenate(
        [floor_position, floor_normal, floor_z_value], axis=-1)[0]
    rel_in = jnp.concatenate(
        [trans_object_obb_center_dist, trans_object_abb_eiou], axis=-1)[0]

    obj_x = _pad2(obj_in, O_NUM, 128)
    wall_x = _pad2(wall_in, W_NUM, 128)
    floor_x = _pad2(floor_in, F_NUM, 128)
    rin3 = _pad2(rel_in, O_NUM * O_NUM, 8).reshape(REAL_BLOCKS, RB * O_NUM, 8)

    ow1 = _pad2(obj_w1, 128, FDIM)
    ww1 = _pad2(wall_w1, 128, FDIM)
    fw1 = _pad2(floor_w1, 128, FDIM)
    rw1 = _pad2(rel_w1, 8, FDIM)

    def row(b):
        return b.reshape(1, -1)

    nsd = jax.ShapeDtypeStruct((T_NUM, FDIM), F32)
    rel_sd = jax.ShapeDtypeStruct((T_NUM * T_NUM, FDIM), BF16)
    wcs_bf = wc_rel_sub.astype(BF16)
    wco_bf = wc_rel_obj.astype(BF16)
    rw2_bf = rel_w2.astype(BF16)

    node_f = pl.pallas_call(_node_enc_kernel, out_shape=nsd)(
        obj_x, ow1, row(obj_b1), obj_w2, row(obj_b2),
        wall_x, ww1, row(wall_b1), wall_w2, row(wall_b2),
        floor_x, fw1, row(floor_b1), floor_w2, row(floor_b2))

    full = lambda shape: pl.BlockSpec(shape, lambda i: (0,) * len(shape))
    blk_spec = pl.BlockSpec((BLK, FDIM), lambda i: (i, 0))
    msub_spec = pl.BlockSpec((RB, FDIM), lambda i: (i, 0))
    scratch = [pltpu.VMEM((T_NUM, FDIM), F32)]

    rel0, msub, mobj = pl.pallas_call(
        _build_reduce_kernel,
        grid=(GRID,),
        in_specs=[
            pl.BlockSpec((1, RB * O_NUM, 8),
                         lambda i: (jnp.minimum(i, REAL_BLOCKS - 1), 0, 0)),
            full((8, FDIM)), full((1, FDIM)), full((FDIM, FDIM)),
            full((1, FDIM)), full((FDIM, FDIM)), full((FDIM, FDIM)),
        ],
        out_specs=[blk_spec, msub_spec, full((T_NUM, FDIM))],
        out_shape=[rel_sd, nsd, nsd],
        scratch_shapes=scratch,
    )(rin3, rw1, row(rel_b1), rw2_bf, row(rel_b2), wcs_bf, wco_bf)

    node_upd = pl.pallas_call(_node_update_kernel,
                              out_shape=[nsd, nsd, nsd])

    def fused(write_rel):
        body = functools.partial(_fused_update_reduce_kernel,
                                 write_rel=write_rel)
        if not write_rel:
            def body(relp, a_ref, b_ref, wcs, wco, msub_ref, mobj_ref,
                     acc_ref):
                _fused_update_reduce_kernel(
                    relp, a_ref, b_ref, wcs, wco, None, msub_ref, mobj_ref,
                    acc_ref, write_rel=False)
        blkf_spec = pl.BlockSpec((BLKF, FDIM), lambda i: (i, 0))
        msubf_spec = pl.BlockSpec((RBF, FDIM), lambda i: (i, 0))
        out_specs = [msubf_spec, full((T_NUM, FDIM))]
        out_shape = [nsd, nsd]
        if write_rel:
            out_specs = [blkf_spec] + out_specs
            out_shape = [rel_sd] + out_shape
        return pl.pallas_call(
            body,
            grid=(GRIDF,),
            in_specs=[blkf_spec, msubf_spec, full((T_NUM, FDIM)),
                      full((FDIM, FDIM)), full((FDIM, FDIM))],
            out_specs=out_specs,
            out_shape=out_shape,
            scratch_shapes=scratch,
        )

    fused_w = fused(True)
    fused_last = fused(False)

    rel = rel0
    for step in range(STEPS):
        node_f, a_vec, b_vec = node_upd(
            node_f, msub, mobj, wu_obj, wc_obj_sub, wc_obj_obj, wu_rel)
        if step == STEPS - 1:
            break
        if step < STEPS - 2:
            rel, msub, mobj = fused_w(rel, a_vec, b_vec, wcs_bf, wco_bf)
        else:
            msub, mobj = fused_last(rel, a_vec, b_vec, wcs_bf, wco_bf)

    osd = jax.ShapeDtypeStruct((O_NUM, 128), F32)
    tr, eu, sc = pl.pallas_call(_out_enc_kernel, out_shape=[osd, osd, osd])(
        node_f,
        tr_w1, row(tr_b1), _pad2(tr_w2, FDIM // 2, 128), row(_pad2(tr_b2.reshape(1, -1), 1, 128)[0]),
        eu_w1, row(eu_b1), _pad2(eu_w2, FDIM // 2, 128), row(_pad2(eu_b2.reshape(1, -1), 1, 128)[0]),
        sc_w1, row(sc_b1), _pad2(sc_w2, FDIM // 2, 128), row(_pad2(sc_b2.reshape(1, -1), 1, 128)[0]))
    return jnp.concatenate([tr[:, :3], eu[:, :3], sc[:, :3]], axis=-1)


# speedup vs baseline: 11.9730x; 1.0199x over previous
import functools

import jax
import jax.numpy as jnp
from jax import lax
from jax.experimental import pallas as pl
from jax.experimental.pallas import tpu as pltpu

O_NUM, W_NUM, F_NUM = 200, 40, 16
T_NUM = O_NUM + W_NUM + F_NUM
FDIM = 512
STEPS = 4
DEG = float(T_NUM - 1)

RB = 8
GRID = T_NUM // RB
BLK = RB * T_NUM
RBF = 16
GRIDF = T_NUM // RBF
BLKF = RBF * T_NUM
REAL_BLOCKS = O_NUM // RB
PAD_VAL = 0.001

F32 = jnp.float32
BF16 = jnp.bfloat16


def _dot(a, b):
    return jnp.dot(a, b, preferred_element_type=F32)


def _node_enc_kernel(ox, ow1, ob1, ow2, ob2,
                     wx, ww1, wb1, ww2, wb2,
                     fx, fw1, fb1, fw2, fb2, out_ref):
    h = jnp.maximum(_dot(ox[...], ow1[...]) + ob1[...], 0.0)
    out_ref[0:O_NUM, :] = _dot(h, ow2[...]) + ob2[...]
    h = jnp.maximum(_dot(wx[...], ww1[...]) + wb1[...], 0.0)
    out_ref[O_NUM:O_NUM + W_NUM, :] = _dot(h, ww2[...]) + wb2[...]
    h = jnp.maximum(_dot(fx[...], fw1[...]) + fb1[...], 0.0)
    out_ref[O_NUM + W_NUM:T_NUM, :] = _dot(h, fw2[...]) + fb2[...]


def _masked_reduce(e2, pid, wcs_ref, wco_ref, asub_ref, aobj_ref, rb, grid):
    p_s = jnp.maximum(_dot(e2, wcs_ref[...]), 0.0)
    p_o = jnp.maximum(_dot(e2, wco_ref[...]), 0.0)
    q = lax.broadcasted_iota(jnp.int32, (rb * T_NUM, FDIM), 0)
    dmask = (q & (T_NUM - 1)) == (q >> 8) + pid * rb
    p_s = jnp.where(dmask, 0.0, p_s)
    p_o = jnp.where(dmask, 0.0, p_o)
    rows = [p_s[r * T_NUM:(r + 1) * T_NUM, :].sum(axis=0, keepdims=True)
            for r in range(rb)]
    asub_ref[pl.ds(pid * rb, rb), :] = jnp.concatenate(rows, axis=0)
    colsum = p_o[0:T_NUM, :]
    for r in range(1, rb):
        colsum = colsum + p_o[r * T_NUM:(r + 1) * T_NUM, :]

    @pl.when(pid == 0)
    def _():
        aobj_ref[...] = colsum

    @pl.when(pid > 0)
    def _():
        aobj_ref[...] = aobj_ref[...] + colsum


def _node_update(asub_ref, aobj_ref, nf_ref, wuo, wcs_o, wco_o, wur,
                 nf_out, a_out, b_out, write_ab):
    m = (asub_ref[...] + aobj_ref[...]) * (0.5 / DEG)
    n2 = jnp.maximum(nf_ref[...] + _dot(m, wuo[...]), 0.0)
    nf_out[...] = n2
    if write_ab:
        a_out[...] = _dot(jnp.maximum(_dot(n2, wcs_o[...]), 0.0),
                          wur[...]) * 0.5
        b_out[...] = _dot(jnp.maximum(_dot(n2, wco_o[...]), 0.0),
                          wur[...]) * 0.5


def _build_reduce_kernel(rin_ref, w1, b1, w2, b2, wcs, wco,
                         nf_ref, wuo, wcs_o, wco_o, wur,
                         rel_ref, nf_out, a_out, b_out,
                         asub_ref, aobj_ref):
    pid = pl.program_id(0)

    @pl.when(pid < REAL_BLOCKS)
    def _():
        x = rin_ref[0]
        h = jnp.maximum(_dot(x, w1[...]) + b1[...], 0.0)
        y = _dot(h.astype(BF16), w2[...]) + b2[...]
        pad = jnp.full((T_NUM - O_NUM, FDIM), PAD_VAL, F32)
        parts = []
        for r in range(RB):
            parts.append(y[r * O_NUM:(r + 1) * O_NUM, :])
            parts.append(pad)
        rel_ref[...] = jnp.concatenate(parts, axis=0).astype(BF16)

    @pl.when(pid >= REAL_BLOCKS)
    def _():
        rel_ref[...] = jnp.full((BLK, FDIM), PAD_VAL, BF16)

    _masked_reduce(rel_ref[...], pid, wcs, wco, asub_ref, aobj_ref,
                   RB, GRID)

    @pl.when(pid == GRID - 1)
    def _():
        _node_update(asub_ref, aobj_ref, nf_ref, wuo, wcs_o, wco_o, wur,
                     nf_out, a_out, b_out, True)


def _fused_kernel(relp, a_ref, b_ref, wcs, wco,
                  nf_ref, wuo, wcs_o, wco_o, wur,
                  reln, nf_out, a_out, b_out, asub_ref, aobj_ref,
                  *, write_rel, write_ab):
    pid = pl.program_id(0)
    e2 = relp[...]
    b = b_ref[...].astype(BF16)
    a = a_ref[...].astype(BF16)
    parts = []
    for r in range(RBF):
        seg = e2[r * T_NUM:(r + 1) * T_NUM, :] + a[r:r + 1, :] + b
        parts.append(jnp.maximum(seg, jnp.bfloat16(0.0)))
    en2 = jnp.concatenate(parts, axis=0)
    if write_rel:
        reln[...] = en2
    _masked_reduce(en2, pid, wcs, wco, asub_ref, aobj_ref, RBF, GRIDF)

    @pl.when(pid == GRIDF - 1)
    def _():
        _node_update(asub_ref, aobj_ref, nf_ref, wuo, wcs_o, wco_o, wur,
                     nf_out, a_out, b_out, write_ab)


def _out_enc_kernel(nf, tw1, tb1, tw2, tb2, ew1, eb1, ew2, eb2,
                    sw1, sb1, sw2, sb2, t_out, e_out, s_out):
    x = nf[0:O_NUM, :]
    for w1, b1, w2, b2, o in ((tw1, tb1, tw2, tb2, t_out),
                              (ew1, eb1, ew2, eb2, e_out),
                              (sw1, sb1, sw2, sb2, s_out)):
        h = _dot(x, w1[...]) + b1[...]
        h = jnp.where(h > 0, h, 0.2 * h)
        o[...] = _dot(h, w2[...]) + b2[...]


def _pad2(x, r, c):
    return jnp.pad(x, ((0, r - x.shape[0]), (0, c - x.shape[1])))


def kernel(floor_position, floor_normal, floor_z_value, wall_position,
           wall_normal, trans_object_obb, trans_object_abb,
           trans_object_obb_center, translate, euler_angle, scale,
           trans_object_obb_center_dist, trans_object_abb_eiou,
           obj_w1, obj_b1, obj_w2, obj_b2,
           rel_w1, rel_b1, rel_w2, rel_b2,
           floor_w1, floor_b1, floor_w2, floor_b2,
           wall_w1, wall_b1, wall_w2, wall_b2,
           wc_rel_sub, wc_rel_obj, wc_obj_sub, wc_obj_obj, wu_obj, wu_rel,
           tr_w1, tr_b1, tr_w2, tr_b2,
           eu_w1, eu_b1, eu_w2, eu_b2,
           sc_w1, sc_b1, sc_w2, sc_b2):
    obj_in = jnp.concatenate(
        [trans_object_obb, trans_object_abb, trans_object_obb_center,
         translate, euler_angle, scale], axis=-1)[0]
    wall_in = jnp.concatenate([wall_position, wall_normal], axis=-1)[0]
    floor_in = jnp.concatenate(
        [floor_position, floor_normal, floor_z_value], axis=-1)[0]
    rel_in = jnp.concatenate(
        [trans_object_obb_center_dist, trans_object_abb_eiou], axis=-1)[0]

    obj_x = _pad2(obj_in, O_NUM, 128)
    wall_x = _pad2(wall_in, W_NUM, 128)
    floor_x = _pad2(floor_in, F_NUM, 128)
    rin3 = _pad2(rel_in, O_NUM * O_NUM, 8).reshape(REAL_BLOCKS, RB * O_NUM, 8)

    ow1 = _pad2(obj_w1, 128, FDIM)
    ww1 = _pad2(wall_w1, 128, FDIM)
    fw1 = _pad2(floor_w1, 128, FDIM)
    rw1 = _pad2(rel_w1, 8, FDIM)

    def row(b):
        return b.reshape(1, -1)

    nsd = jax.ShapeDtypeStruct((T_NUM, FDIM), F32)
    rel_sd = jax.ShapeDtypeStruct((T_NUM * T_NUM, FDIM), BF16)
    wcs_bf = wc_rel_sub.astype(BF16)
    wco_bf = wc_rel_obj.astype(BF16)
    rw2_bf = rel_w2.astype(BF16)

    node_f = pl.pallas_call(_node_enc_kernel, out_shape=nsd)(
        obj_x, ow1, row(obj_b1), obj_w2, row(obj_b2),
        wall_x, ww1, row(wall_b1), wall_w2, row(wall_b2),
        floor_x, fw1, row(floor_b1), floor_w2, row(floor_b2))

    full = lambda shape: pl.BlockSpec(shape, lambda i: (0,) * len(shape))
    blk_spec = pl.BlockSpec((BLK, FDIM), lambda i: (i, 0))
    scratch = [pltpu.VMEM((T_NUM, FDIM), F32), pltpu.VMEM((T_NUM, FDIM), F32)]
    wspecs = [full((T_NUM, FDIM))] + [full((FDIM, FDIM))] * 4

    rel0, nf1, a1, b1 = pl.pallas_call(
        _build_reduce_kernel,
        grid=(GRID,),
        in_specs=[
            pl.BlockSpec((1, RB * O_NUM, 8),
                         lambda i: (jnp.minimum(i, REAL_BLOCKS - 1), 0, 0)),
            full((8, FDIM)), full((1, FDIM)), full((FDIM, FDIM)),
            full((1, FDIM)), full((FDIM, FDIM)), full((FDIM, FDIM)),
        ] + wspecs,
        out_specs=[blk_spec, full((T_NUM, FDIM)), full((T_NUM, FDIM)),
                   full((T_NUM, FDIM))],
        out_shape=[rel_sd, nsd, nsd, nsd],
        scratch_shapes=scratch,
    )(rin3, rw1, row(rel_b1), rw2_bf, row(rel_b2), wcs_bf, wco_bf,
      node_f, wu_obj, wc_obj_sub, wc_obj_obj, wu_rel)

    blkf_spec = pl.BlockSpec((BLKF, FDIM), lambda i: (i, 0))
    msubf_spec = pl.BlockSpec((RBF, FDIM), lambda i: (i, 0))

    def fused(write_rel, write_ab):
        body = functools.partial(_fused_kernel, write_rel=write_rel,
                                 write_ab=write_ab)
        if not write_rel:
            def body(relp, a_ref, b_ref, wcs, wco, nf_ref, wuo, wcs_o,
                     wco_o, wur, nf_out, asub_ref, aobj_ref):
                _fused_kernel(relp, a_ref, b_ref, wcs, wco, nf_ref, wuo,
                              wcs_o, wco_o, wur, None, nf_out, None, None,
                              asub_ref, aobj_ref,
                              write_rel=False, write_ab=False)
        out_specs = [full((T_NUM, FDIM))]
        out_shape = [nsd]
        if write_rel:
            out_specs = [blkf_spec] + out_specs \
                + [full((T_NUM, FDIM)), full((T_NUM, FDIM))]
            out_shape = [rel_sd] + out_shape + [nsd, nsd]
        return pl.pallas_call(
            body,
            grid=(GRIDF,),
            in_specs=[blkf_spec, msubf_spec, full((T_NUM, FDIM)),
                      full((FDIM, FDIM)), full((FDIM, FDIM))] + wspecs,
            out_specs=out_specs,
            out_shape=out_shape,
            scratch_shapes=scratch,
        )

    fused_w = fused(True, True)
    fused_last = fused(False, False)

    rel1, nf2, a2, b2 = fused_w(rel0, a1, b1, wcs_bf, wco_bf,
                                nf1, wu_obj, wc_obj_sub, wc_obj_obj, wu_rel)
    rel2, nf3, a3, b3 = fused_w(rel1, a2, b2, wcs_bf, wco_bf,
                                nf2, wu_obj, wc_obj_sub, wc_obj_obj, wu_rel)
    nf4 = fused_last(rel2, a3, b3, wcs_bf, wco_bf,
                     nf3, wu_obj, wc_obj_sub, wc_obj_obj, wu_rel)[0]

    osd = jax.ShapeDtypeStruct((O_NUM, 128), F32)
    tr, eu, sc = pl.pallas_call(_out_enc_kernel, out_shape=[osd, osd, osd])(
        nf4,
        tr_w1, row(tr_b1), _pad2(tr_w2, FDIM // 2, 128), row(_pad2(tr_b2.reshape(1, -1), 1, 128)[0]),
        eu_w1, row(eu_b1), _pad2(eu_w2, FDIM // 2, 128), row(_pad2(eu_b2.reshape(1, -1), 1, 128)[0]),
        sc_w1, row(sc_b1), _pad2(sc_w2, FDIM // 2, 128), row(_pad2(sc_b2.reshape(1, -1), 1, 128)[0]))
    return jnp.concatenate([tr[:, :3], eu[:, :3], sc[:, :3]], axis=-1)


# speedup vs baseline: 11.9837x; 1.0009x over previous
import functools

import jax
import jax.numpy as jnp
from jax import lax
from jax.experimental import pallas as pl
from jax.experimental.pallas import tpu as pltpu

O_NUM, W_NUM, F_NUM = 200, 40, 16
T_NUM = O_NUM + W_NUM + F_NUM
FDIM = 512
STEPS = 4
DEG = float(T_NUM - 1)

RB = 8
GRID = T_NUM // RB
BLK = RB * T_NUM
RBF = 16
GRIDF = T_NUM // RBF
BLKF = RBF * T_NUM
REAL_BLOCKS = O_NUM // RB
PAD_VAL = 0.001

F32 = jnp.float32
BF16 = jnp.bfloat16


def _dot(a, b):
    return jnp.dot(a, b, preferred_element_type=F32)


def _node_enc_kernel(ox, ow1, ob1, ow2, ob2,
                     wx, ww1, wb1, ww2, wb2,
                     fx, fw1, fb1, fw2, fb2, out_ref):
    h = jnp.maximum(_dot(ox[...], ow1[...]) + ob1[...], 0.0)
    out_ref[0:O_NUM, :] = _dot(h, ow2[...]) + ob2[...]
    h = jnp.maximum(_dot(wx[...], ww1[...]) + wb1[...], 0.0)
    out_ref[O_NUM:O_NUM + W_NUM, :] = _dot(h, ww2[...]) + wb2[...]
    h = jnp.maximum(_dot(fx[...], fw1[...]) + fb1[...], 0.0)
    out_ref[O_NUM + W_NUM:T_NUM, :] = _dot(h, fw2[...]) + fb2[...]


def _masked_reduce(e2, pid, wcat_ref, asub_ref, aobj_ref, rb, grid):
    p = jnp.maximum(_dot(e2, wcat_ref[...]), 0.0)
    q = lax.broadcasted_iota(jnp.int32, (rb * T_NUM, 2 * FDIM), 0)
    dmask = (q & (T_NUM - 1)) == (q >> 8) + pid * rb
    p = jnp.where(dmask, 0.0, p)
    p_s = p[:, :FDIM]
    p_o = p[:, FDIM:]
    rows = [p_s[r * T_NUM:(r + 1) * T_NUM, :].sum(axis=0, keepdims=True)
            for r in range(rb)]
    asub_ref[pl.ds(pid * rb, rb), :] = jnp.concatenate(rows, axis=0)
    colsum = p_o[0:T_NUM, :]
    for r in range(1, rb):
        colsum = colsum + p_o[r * T_NUM:(r + 1) * T_NUM, :]

    @pl.when(pid == 0)
    def _():
        aobj_ref[...] = colsum

    @pl.when(pid > 0)
    def _():
        aobj_ref[...] = aobj_ref[...] + colsum


def _node_update(asub_ref, aobj_ref, nf_ref, wuo, wcs_o, wco_o, wur,
                 nf_out, a_out, b_out, write_ab):
    m = (asub_ref[...] + aobj_ref[...]) * (0.5 / DEG)
    n2 = jnp.maximum(nf_ref[...] + _dot(m, wuo[...]), 0.0)
    nf_out[...] = n2
    if write_ab:
        a_out[...] = _dot(jnp.maximum(_dot(n2, wcs_o[...]), 0.0),
                          wur[...]) * 0.5
        b_out[...] = _dot(jnp.maximum(_dot(n2, wco_o[...]), 0.0),
                          wur[...]) * 0.5


def _build_reduce_kernel(rin_ref, w1, b1, w2, b2, wcat,
                         nf_ref, wuo, wcs_o, wco_o, wur,
                         rel_ref, nf_out, a_out, b_out,
                         asub_ref, aobj_ref):
    pid = pl.program_id(0)

    @pl.when(pid < REAL_BLOCKS)
    def _():
        x = rin_ref[0]
        h = jnp.maximum(_dot(x, w1[...]) + b1[...], 0.0)
        y = _dot(h.astype(BF16), w2[...]) + b2[...]
        pad = jnp.full((T_NUM - O_NUM, FDIM), PAD_VAL, F32)
        parts = []
        for r in range(RB):
            parts.append(y[r * O_NUM:(r + 1) * O_NUM, :])
            parts.append(pad)
        rel_ref[...] = jnp.concatenate(parts, axis=0).astype(BF16)

    @pl.when(pid >= REAL_BLOCKS)
    def _():
        rel_ref[...] = jnp.full((BLK, FDIM), PAD_VAL, BF16)

    _masked_reduce(rel_ref[...], pid, wcat, asub_ref, aobj_ref,
                   RB, GRID)

    @pl.when(pid == GRID - 1)
    def _():
        _node_update(asub_ref, aobj_ref, nf_ref, wuo, wcs_o, wco_o, wur,
                     nf_out, a_out, b_out, True)


def _fused_kernel(relp, a_ref, b_ref, wcat,
                  nf_ref, wuo, wcs_o, wco_o, wur,
                  reln, nf_out, a_out, b_out, asub_ref, aobj_ref,
                  *, write_rel, write_ab):
    pid = pl.program_id(0)
    e2 = relp[...]
    b = b_ref[...].astype(BF16)
    a = a_ref[...].astype(BF16)
    parts = []
    for r in range(RBF):
        seg = e2[r * T_NUM:(r + 1) * T_NUM, :] + a[r:r + 1, :] + b
        parts.append(jnp.maximum(seg, jnp.bfloat16(0.0)))
    en2 = jnp.concatenate(parts, axis=0)
    if write_rel:
        reln[...] = en2
    _masked_reduce(en2, pid, wcat, asub_ref, aobj_ref, RBF, GRIDF)

    @pl.when(pid == GRIDF - 1)
    def _():
        _node_update(asub_ref, aobj_ref, nf_ref, wuo, wcs_o, wco_o, wur,
                     nf_out, a_out, b_out, write_ab)


def _out_enc_kernel(nf, tw1, tb1, tw2, tb2, ew1, eb1, ew2, eb2,
                    sw1, sb1, sw2, sb2, t_out, e_out, s_out):
    x = nf[0:O_NUM, :]
    for w1, b1, w2, b2, o in ((tw1, tb1, tw2, tb2, t_out),
                              (ew1, eb1, ew2, eb2, e_out),
                              (sw1, sb1, sw2, sb2, s_out)):
        h = _dot(x, w1[...]) + b1[...]
        h = jnp.where(h > 0, h, 0.2 * h)
        o[...] = _dot(h, w2[...]) + b2[...]


def _pad2(x, r, c):
    return jnp.pad(x, ((0, r - x.shape[0]), (0, c - x.shape[1])))


def kernel(floor_position, floor_normal, floor_z_value, wall_position,
           wall_normal, trans_object_obb, trans_object_abb,
           trans_object_obb_center, translate, euler_angle, scale,
           trans_object_obb_center_dist, trans_object_abb_eiou,
           obj_w1, obj_b1, obj_w2, obj_b2,
           rel_w1, rel_b1, rel_w2, rel_b2,
           floor_w1, floor_b1, floor_w2, floor_b2,
           wall_w1, wall_b1, wall_w2, wall_b2,
           wc_rel_sub, wc_rel_obj, wc_obj_sub, wc_obj_obj, wu_obj, wu_rel,
           tr_w1, tr_b1, tr_w2, tr_b2,
           eu_w1, eu_b1, eu_w2, eu_b2,
           sc_w1, sc_b1, sc_w2, sc_b2):
    obj_in = jnp.concatenate(
        [trans_object_obb, trans_object_abb, trans_object_obb_center,
         translate, euler_angle, scale], axis=-1)[0]
    wall_in = jnp.concatenate([wall_position, wall_normal], axis=-1)[0]
    floor_in = jnp.concatenate(
        [floor_position, floor_normal, floor_z_value], axis=-1)[0]
    rel_in = jnp.concatenate(
        [trans_object_obb_center_dist, trans_object_abb_eiou], axis=-1)[0]

    obj_x = _pad2(obj_in, O_NUM, 128)
    wall_x = _pad2(wall_in, W_NUM, 128)
    floor_x = _pad2(floor_in, F_NUM, 128)
    rin3 = _pad2(rel_in, O_NUM * O_NUM, 8).reshape(REAL_BLOCKS, RB * O_NUM, 8)

    ow1 = _pad2(obj_w1, 128, FDIM)
    ww1 = _pad2(wall_w1, 128, FDIM)
    fw1 = _pad2(floor_w1, 128, FDIM)
    rw1 = _pad2(rel_w1, 8, FDIM)

    def row(b):
        return b.reshape(1, -1)

    nsd = jax.ShapeDtypeStruct((T_NUM, FDIM), F32)
    rel_sd = jax.ShapeDtypeStruct((T_NUM * T_NUM, FDIM), BF16)
    wcat_bf = jnp.concatenate([wc_rel_sub, wc_rel_obj], axis=1).astype(BF16)
    rw2_bf = rel_w2.astype(BF16)

    node_f = pl.pallas_call(_node_enc_kernel, out_shape=nsd)(
        obj_x, ow1, row(obj_b1), obj_w2, row(obj_b2),
        wall_x, ww1, row(wall_b1), wall_w2, row(wall_b2),
        floor_x, fw1, row(floor_b1), floor_w2, row(floor_b2))

    full = lambda shape: pl.BlockSpec(shape, lambda i: (0,) * len(shape))
    blk_spec = pl.BlockSpec((BLK, FDIM), lambda i: (i, 0))
    scratch = [pltpu.VMEM((T_NUM, FDIM), F32), pltpu.VMEM((T_NUM, FDIM), F32)]
    wspecs = [full((T_NUM, FDIM))] + [full((FDIM, FDIM))] * 4

    rel0, nf1, a1, b1 = pl.pallas_call(
        _build_reduce_kernel,
        grid=(GRID,),
        in_specs=[
            pl.BlockSpec((1, RB * O_NUM, 8),
                         lambda i: (jnp.minimum(i, REAL_BLOCKS - 1), 0, 0)),
            full((8, FDIM)), full((1, FDIM)), full((FDIM, FDIM)),
            full((1, FDIM)), full((FDIM, 2 * FDIM)),
        ] + wspecs,
        out_specs=[blk_spec, full((T_NUM, FDIM)), full((T_NUM, FDIM)),
                   full((T_NUM, FDIM))],
        out_shape=[rel_sd, nsd, nsd, nsd],
        scratch_shapes=scratch,
    )(rin3, rw1, row(rel_b1), rw2_bf, row(rel_b2), wcat_bf,
      node_f, wu_obj, wc_obj_sub, wc_obj_obj, wu_rel)

    blkf_spec = pl.BlockSpec((BLKF, FDIM), lambda i: (i, 0))
    msubf_spec = pl.BlockSpec((RBF, FDIM), lambda i: (i, 0))

    def fused(write_rel, write_ab):
        body = functools.partial(_fused_kernel, write_rel=write_rel,
                                 write_ab=write_ab)
        if not write_rel:
            def body(relp, a_ref, b_ref, wcat, nf_ref, wuo, wcs_o,
                     wco_o, wur, nf_out, asub_ref, aobj_ref):
                _fused_kernel(relp, a_ref, b_ref, wcat, nf_ref, wuo,
                              wcs_o, wco_o, wur, None, nf_out, None, None,
                              asub_ref, aobj_ref,
                              write_rel=False, write_ab=False)
        out_specs = [full((T_NUM, FDIM))]
        out_shape = [nsd]
        if write_rel:
            out_specs = [blkf_spec] + out_specs \
                + [full((T_NUM, FDIM)), full((T_NUM, FDIM))]
            out_shape = [rel_sd] + out_shape + [nsd, nsd]
        return pl.pallas_call(
            body,
            grid=(GRIDF,),
            in_specs=[blkf_spec, msubf_spec, full((T_NUM, FDIM)),
                      full((FDIM, 2 * FDIM))] + wspecs,
            out_specs=out_specs,
            out_shape=out_shape,
            scratch_shapes=scratch,
        )

    fused_w = fused(True, True)
    fused_last = fused(False, False)

    rel1, nf2, a2, b2 = fused_w(rel0, a1, b1, wcat_bf,
                                nf1, wu_obj, wc_obj_sub, wc_obj_obj, wu_rel)
    rel2, nf3, a3, b3 = fused_w(rel1, a2, b2, wcat_bf,
                                nf2, wu_obj, wc_obj_sub, wc_obj_obj, wu_rel)
    nf4 = fused_last(rel2, a3, b3, wcat_bf,
                     nf3, wu_obj, wc_obj_sub, wc_obj_obj, wu_rel)[0]

    osd = jax.ShapeDtypeStruct((O_NUM, 128), F32)
    tr, eu, sc = pl.pallas_call(_out_enc_kernel, out_shape=[osd, osd, osd])(
        nf4,
        tr_w1, row(tr_b1), _pad2(tr_w2, FDIM // 2, 128), row(_pad2(tr_b2.reshape(1, -1), 1, 128)[0]),
        eu_w1, row(eu_b1), _pad2(eu_w2, FDIM // 2, 128), row(_pad2(eu_b2.reshape(1, -1), 1, 128)[0]),
        sc_w1, row(sc_b1), _pad2(sc_w2, FDIM // 2, 128), row(_pad2(sc_b2.reshape(1, -1), 1, 128)[0]))
    return jnp.concatenate([tr[:, :3], eu[:, :3], sc[:, :3]], axis=-1)
